# baseline (speedup 1.0000x reference)
import functools

import jax
import jax.numpy as jnp
from jax import lax
from jax.experimental import pallas as pl
from jax.experimental.pallas import tpu as pltpu
from jax.experimental.pallas import tpu_sc as plsc

WH, WW = 16, 16
NUM_HEADS = 16
N = WH * WW
B = N * N
ROWS = (2 * WH - 1) * (2 * WW - 1)

_L = 16
_NW = 32
_BPW = B // _NW
_RPW = _BPW // N
_HALF_R = _RPW // 2
_GROUPS = _BPW // _L

_mesh = plsc.VectorSubcoreMesh(core_axis_name="c", subcore_axis_name="s")


@functools.partial(
    pl.kernel,
    mesh=_mesh,
    compiler_params=pltpu.CompilerParams(needs_layout_passes=False),
    out_type=jax.ShapeDtypeStruct((1, NUM_HEADS, N, N), jnp.float32),
    scratch_types=[
        pltpu.VMEM((ROWS * NUM_HEADS,), jnp.float32),
        pltpu.VMEM((_BPW,), jnp.int32),
        pltpu.VMEM((NUM_HEADS, _RPW, N), jnp.float32),
        pltpu.SemaphoreType.DMA,
        pltpu.SemaphoreType.DMA,
    ],
)
def _bias_gather(table_hbm, idx_hbm, out_hbm, table_v, idx_v, out_v,
                 in_sem, out_sem):
    wid = lax.axis_index("s") * 2 + lax.axis_index("c")
    base = wid * _BPW

    ct = pltpu.async_copy(table_hbm, table_v, in_sem)
    ci = pltpu.async_copy(idx_hbm.at[pl.ds(base, _BPW)], idx_v, in_sem)
    ct.wait()
    ci.wait()

    def run_chunk(k, carry):
        lo = k * (_GROUPS // 8)

        @plsc.parallel_loop(lo, lo + _GROUPS // 8, unroll=1)
        def _(g):
            iv = idx_v[pl.ds(g * _L, _L)]
            r = g // _L
            c = (g % _L) * _L
            rowbase = iv * NUM_HEADS
            vals = [plsc.load_gather(table_v, [rowbase + h])
                    for h in range(NUM_HEADS)]
            for h in range(NUM_HEADS):
                out_v[h, r, pl.ds(c, _L)] = vals[h]

        r0 = k * (_RPW // 8)
        pltpu.async_copy(
            out_v.at[:, pl.ds(r0, _RPW // 8)],
            out_hbm.at[0, :, pl.ds(wid * _RPW + r0, _RPW // 8), :],
            out_sem)
        return carry

    lax.fori_loop(0, 8, run_chunk, 0)
    pltpu.make_async_copy(
        out_v, out_hbm.at[0, :, pl.ds(wid * _RPW, _RPW), :], out_sem).wait()


def kernel(relative_position_bias_table, relative_position_index):
    return _bias_gather(relative_position_bias_table.reshape(-1),
                        relative_position_index)


# speedup vs baseline: 1.0166x; 1.0166x over previous
import functools

import jax
import jax.numpy as jnp
from jax import lax
from jax.experimental import pallas as pl
from jax.experimental.pallas import tpu as pltpu
from jax.experimental.pallas import tpu_sc as plsc

WH, WW = 16, 16
NUM_HEADS = 16
N = WH * WW
B = N * N
ROWS = (2 * WH - 1) * (2 * WW - 1)

_L = 16
_NW = 32
_BPW = B // _NW
_RPW = _BPW // N
_GROUPS = _BPW // _L

_mesh = plsc.VectorSubcoreMesh(core_axis_name="c", subcore_axis_name="s")


@functools.partial(
    pl.kernel,
    mesh=_mesh,
    compiler_params=pltpu.CompilerParams(needs_layout_passes=False),
    out_type=jax.ShapeDtypeStruct((1, NUM_HEADS, N, N), jnp.float32),
    scratch_types=[
        pltpu.VMEM((ROWS * NUM_HEADS,), jnp.float32),
        pltpu.VMEM((_BPW,), jnp.int32),
        pltpu.VMEM((NUM_HEADS, _RPW, N), jnp.float32),
        pltpu.SemaphoreType.DMA,
        pltpu.SemaphoreType.DMA,
    ],
)
def _bias_gather(table_hbm, idx_hbm, out_hbm, table_v, idx_v, out_v,
                 in_sem, out_sem):
    wid = lax.axis_index("s") * 2 + lax.axis_index("c")
    base = wid * _BPW

    ct = pltpu.async_copy(table_hbm, table_v, in_sem)
    ci = pltpu.async_copy(idx_hbm.at[pl.ds(base, _BPW)], idx_v, in_sem)
    ct.wait()
    ci.wait()

    def run_chunk(k, carry):
        lo = k * (_GROUPS // 4)

        @plsc.parallel_loop(lo, lo + _GROUPS // 4, unroll=1)
        def _(g):
            iv = idx_v[pl.ds(g * _L, _L)]
            r = g // _L
            c = (g % _L) * _L
            rowbase = iv * NUM_HEADS
            vals = [plsc.load_gather(table_v, [rowbase + h])
                    for h in range(NUM_HEADS)]
            for h in range(NUM_HEADS):
                out_v[h, r, pl.ds(c, _L)] = vals[h]

        r0 = k * (_RPW // 4)
        pltpu.async_copy(
            out_v.at[:, pl.ds(r0, _RPW // 4)],
            out_hbm.at[0, :, pl.ds(wid * _RPW + r0, _RPW // 4), :],
            out_sem)
        return carry

    lax.fori_loop(0, 4, run_chunk, 0)
    pltpu.make_async_copy(
        out_v, out_hbm.at[0, :, pl.ds(wid * _RPW, _RPW), :], out_sem).wait()


def kernel(relative_position_bias_table, relative_position_index):
    return _bias_gather(relative_position_bias_table.reshape(-1),
                        relative_position_index)
